# baseline (speedup 1.0000x reference)
import math

import jax
import jax.numpy as jnp
from jax import lax
from jax.experimental import pallas as pl
from jax.experimental.pallas import tpu as pltpu
from jax.experimental.pallas import tpu_sc as plsc

D_MODEL = 64
SCALE = math.sqrt(D_MODEL)

_NC = 2
_NS = 16
_NW = _NC * _NS

_RCHUNK = 256
_NBUF = 2


def _emb_body(table_hbm, xt_hbm, out_hbm, idx_v, *scratch):
    n_cols = xt_hbm.shape[0]
    rows = scratch[0:_NBUF]
    trows = scratch[_NBUF:2 * _NBUF]
    gsem = scratch[2 * _NBUF:3 * _NBUF]
    ssem = scratch[3 * _NBUF:4 * _NBUF]

    wid = lax.axis_index("s") * _NC + lax.axis_index("c")
    rpw = idx_v.shape[1]
    r0 = wid * rpw
    pltpu.sync_copy(xt_hbm.at[:, pl.ds(r0, rpw)], idx_v)

    halves = rpw // _RCHUNK
    nchunk = n_cols * halves

    def start_gather(t, b):
        c = t // halves
        h = t % halves
        pltpu.async_copy(
            table_hbm.at[idx_v.at[c, pl.ds(h * _RCHUNK, _RCHUNK)]],
            rows[b], gsem[b],
        )

    def wait_gather(t, b):
        c = t // halves
        h = t % halves
        pltpu.make_async_copy(
            table_hbm.at[idx_v.at[c, pl.ds(h * _RCHUNK, _RCHUNK)]],
            rows[b], gsem[b],
        ).wait()

    def out_slice(t):
        c = t // halves
        h = t % halves
        rt = (r0 + h * _RCHUNK) // 128
        return out_hbm.at[c, :, pl.ds(rt, _RCHUNK // 128)]

    def tr_slice(b):
        return trows[b].at[:, :, pl.ds(0, 8), pl.ds(0, 128)]

    def start_scatter(t, b):
        pltpu.async_copy(tr_slice(b), out_slice(t), ssem[b])

    def wait_scatter(t, b):
        pltpu.make_async_copy(tr_slice(b), out_slice(t), ssem[b]).wait()

    iota = lax.iota(jnp.int32, 16)
    jt_c = [2 * k + iota // 8 for k in range(D_MODEL // 16)]
    js_c = iota % 8

    def transpose_scale(b):
        src, dst = rows[b], trows[b]

        @plsc.parallel_loop(0, 128, 1, unroll=2)
        def tok_body(tl):
            rl_v = jnp.full((16,), tl, dtype=jnp.int32)
            for rt in range(_RCHUNK // 128):
                rt_v = jnp.full((16,), rt, dtype=jnp.int32)
                t = rt * 128 + tl
                for k in range(D_MODEL // 16):
                    v = src[t, pl.ds(16 * k, 16)]
                    plsc.store_scatter(
                        dst, [jt_c[k], rt_v, js_c, rl_v], v * SCALE
                    )

    for b in range(_NBUF):
        start_gather(b, b)

    def group(g, carry):
        t0 = g * _NBUF
        for b in range(_NBUF):
            wait_gather(t0 + b, b)
            transpose_scale(b)
            start_scatter(t0 + b, b)
        for b in range(_NBUF):
            wait_scatter(t0 + b, b)
            start_gather(t0 + b + _NBUF, b)
        return carry

    ngroups = nchunk // _NBUF
    lax.fori_loop(0, ngroups - 1, group, 0)

    t0 = (ngroups - 1) * _NBUF
    for b in range(_NBUF):
        wait_gather(t0 + b, b)
        transpose_scale(b)
        start_scatter(t0 + b, b)
    for b in range(_NBUF):
        wait_scatter(t0 + b, b)


_TCH = 64
_TNBUF = 2


def _tr_body(tt2_hbm, out_hbm, *scratch):
    vocab = out_hbm.shape[0]
    inb = scratch[0:_TNBUF]
    outb = scratch[_TNBUF:2 * _TNBUF]
    idxb = scratch[2 * _TNBUF:3 * _TNBUF]
    gsem = scratch[3 * _TNBUF:4 * _TNBUF]
    ssem = scratch[4 * _TNBUF:5 * _TNBUF]

    wid = lax.axis_index("s") * _NC + lax.axis_index("c")
    nch = vocab // _TCH
    base = nch // _NW * _NW
    nrounds = nch // _NW

    iota16 = lax.iota(jnp.int32, 16)

    def t0_of(i):
        return (i * _NW + wid) * _TCH

    def start_gather(i, b):
        ci = i * _NW + wid
        for m in range(D_MODEL // 16):
            idxb[b][pl.ds(16 * m, 16)] = (iota16 + 16 * m) * nch + ci
        pltpu.async_copy(tt2_hbm.at[idxb[b]], inb[b], gsem[b])

    def wait_gather(i, b):
        pltpu.make_async_copy(
            tt2_hbm.at[idxb[b]], inb[b], gsem[b]
        ).wait()

    def start_scatter(i, b):
        pltpu.async_copy(
            outb[b].at[:, pl.ds(0, D_MODEL)],
            out_hbm.at[pl.ds(t0_of(i), _TCH)], ssem[b],
        )

    def wait_scatter(i, b):
        pltpu.make_async_copy(
            outb[b].at[:, pl.ds(0, D_MODEL)],
            out_hbm.at[pl.ds(t0_of(i), _TCH)], ssem[b],
        ).wait()

    iota = lax.iota(jnp.int32, 16)

    def transpose(b):
        src, dst = inb[b], outb[b]

        @plsc.parallel_loop(0, D_MODEL, 1, unroll=2)
        def j_body(j):
            jv = jnp.full((16,), j, dtype=jnp.int32)
            for m in range(_TCH // 16):
                v = src[j, pl.ds(16 * m, 16)]
                plsc.store_scatter(dst, [iota + 16 * m, jv], v)

    for b in range(_TNBUF):
        start_gather(b, b)

    def group(g, carry):
        i0 = g * _TNBUF
        for b in range(_TNBUF):
            wait_gather(i0 + b, b)
            transpose(b)
            start_scatter(i0 + b, b)
        for b in range(_TNBUF):
            wait_scatter(i0 + b, b)
            start_gather(i0 + b + _TNBUF, b)
        return carry

    ngroups = nrounds // _TNBUF
    lax.fori_loop(0, ngroups - 1, group, 0)

    i0 = (ngroups - 1) * _TNBUF
    for b in range(_TNBUF):
        wait_gather(i0 + b, b)
        transpose(b)
        start_scatter(i0 + b, b)
    for b in range(_TNBUF):
        wait_scatter(i0 + b, b)

    rem = nch - base

    @pl.when(wid < rem)
    def _tail():
        ci = base + wid
        t0 = ci * _TCH
        for m in range(D_MODEL // 16):
            idxb[0][pl.ds(16 * m, 16)] = (iota16 + 16 * m) * nch + ci
        pltpu.async_copy(tt2_hbm.at[idxb[0]], inb[0], gsem[0])
        pltpu.make_async_copy(tt2_hbm.at[idxb[0]], inb[0], gsem[0]).wait()
        transpose(0)
        pltpu.sync_copy(
            outb[0].at[:, pl.ds(0, D_MODEL)], out_hbm.at[pl.ds(t0, _TCH)]
        )


@jax.jit
def kernel(x, table):
    n_rows, n_cols = x.shape
    rpw = n_rows // _NW
    nrt = n_rows // 128
    xt = x.astype(jnp.int32).T
    vocab = table.shape[0]
    tt2 = table.T.reshape(vocab, D_MODEL)

    mesh = plsc.VectorSubcoreMesh(core_axis_name="c", subcore_axis_name="s")
    tr_scratch = (
        [pltpu.VMEM((D_MODEL, _TCH), jnp.float32) for _ in range(_TNBUF)]
        + [pltpu.VMEM((_TCH, D_MODEL + 1), jnp.float32) for _ in range(_TNBUF)]
        + [pltpu.VMEM((D_MODEL,), jnp.int32) for _ in range(_TNBUF)]
        + [pltpu.SemaphoreType.DMA for _ in range(2 * _TNBUF)]
    )
    tr_fn = pl.kernel(
        _tr_body,
        out_type=jax.ShapeDtypeStruct((vocab, D_MODEL), jnp.float32),
        mesh=mesh,
        scratch_types=tr_scratch,
        compiler_params=pltpu.CompilerParams(
            use_tc_tiling_on_sc=False, needs_layout_passes=False
        ),
    )
    table_rm = tr_fn(tt2)

    scratch = (
        [pltpu.VMEM((n_cols, rpw), jnp.int32)]
        + [pltpu.VMEM((_RCHUNK, D_MODEL), jnp.float32) for _ in range(_NBUF)]
        + [
            pltpu.VMEM((8, _RCHUNK // 128, 12, 129), jnp.float32)
            for _ in range(_NBUF)
        ]
        + [pltpu.SemaphoreType.DMA for _ in range(2 * _NBUF)]
    )
    fn = pl.kernel(
        _emb_body,
        out_type=jax.ShapeDtypeStruct((n_cols, 8, nrt, 8, 128), jnp.float32),
        mesh=mesh,
        scratch_types=scratch,
        compiler_params=pltpu.CompilerParams(
            use_tc_tiling_on_sc=False, needs_layout_passes=False
        ),
    )
    o5 = fn(table_rm, xt)
    out_t = o5.transpose(0, 1, 3, 2, 4).reshape(n_cols, D_MODEL, n_rows)
    return out_t.transpose(2, 0, 1)


# speedup vs baseline: 7.0033x; 7.0033x over previous
import math

import jax
import jax.numpy as jnp
from jax import lax
from jax.experimental import pallas as pl
from jax.experimental.pallas import tpu as pltpu
from jax.experimental.pallas import tpu_sc as plsc

D_MODEL = 64
SCALE = math.sqrt(D_MODEL)

_NC = 2
_NS = 16
_NW = _NC * _NS

_RCHUNK = 256
_NBUF = 2


def _emb_body(table_hbm, xt_hbm, out_hbm, idx_v, *scratch):
    n_cols = xt_hbm.shape[0]
    rows = scratch[0:_NBUF]
    trows = scratch[_NBUF:2 * _NBUF]
    gsem = scratch[2 * _NBUF:3 * _NBUF]
    ssem = scratch[3 * _NBUF:4 * _NBUF]

    wid = lax.axis_index("s") * _NC + lax.axis_index("c")
    rpw = idx_v.shape[1]
    r0 = wid * rpw
    pltpu.sync_copy(xt_hbm.at[:, pl.ds(r0, rpw)], idx_v)

    halves = rpw // _RCHUNK
    nchunk = n_cols * halves

    def start_gather(t, b):
        c = t // halves
        h = t % halves
        pltpu.async_copy(
            table_hbm.at[idx_v.at[c, pl.ds(h * _RCHUNK, _RCHUNK)]],
            rows[b], gsem[b],
        )

    def wait_gather(t, b):
        c = t // halves
        h = t % halves
        pltpu.make_async_copy(
            table_hbm.at[idx_v.at[c, pl.ds(h * _RCHUNK, _RCHUNK)]],
            rows[b], gsem[b],
        ).wait()

    def out_slice(t):
        c = t // halves
        h = t % halves
        rt = (r0 + h * _RCHUNK) // 128
        return out_hbm.at[c, :, pl.ds(rt, _RCHUNK // 128)]

    def tr_slice(b):
        return trows[b].at[:, :, pl.ds(0, 8), pl.ds(0, 128)]

    def start_scatter(t, b):
        pltpu.async_copy(tr_slice(b), out_slice(t), ssem[b])

    def wait_scatter(t, b):
        pltpu.make_async_copy(tr_slice(b), out_slice(t), ssem[b]).wait()

    iota = lax.iota(jnp.int32, 16)
    jt_c = [2 * k + iota // 8 for k in range(D_MODEL // 16)]
    js_c = iota % 8

    def transpose_scale(b):
        src, dst = rows[b], trows[b]

        @plsc.parallel_loop(0, 128, 1, unroll=2)
        def tok_body(tl):
            rl_v = jnp.full((16,), tl, dtype=jnp.int32)
            for rt in range(_RCHUNK // 128):
                rt_v = jnp.full((16,), rt, dtype=jnp.int32)
                t = rt * 128 + tl
                for k in range(D_MODEL // 16):
                    v = src[t, pl.ds(16 * k, 16)]
                    plsc.store_scatter(
                        dst, [jt_c[k], rt_v, js_c, rl_v], v * SCALE
                    )

    for b in range(_NBUF):
        start_gather(b, b)

    def group(g, carry):
        t0 = g * _NBUF
        for b in range(_NBUF):
            wait_gather(t0 + b, b)
            transpose_scale(b)
            start_scatter(t0 + b, b)
        for b in range(_NBUF):
            wait_scatter(t0 + b, b)
            start_gather(t0 + b + _NBUF, b)
        return carry

    ngroups = nchunk // _NBUF
    lax.fori_loop(0, ngroups - 1, group, 0)

    t0 = (ngroups - 1) * _NBUF
    for b in range(_NBUF):
        wait_gather(t0 + b, b)
        transpose_scale(b)
        start_scatter(t0 + b, b)
    for b in range(_NBUF):
        wait_scatter(t0 + b, b)


@jax.jit
def kernel(x, table):
    n_rows, n_cols = x.shape
    rpw = n_rows // _NW
    nrt = n_rows // 128
    xt = x.astype(jnp.int32).T

    mesh = plsc.VectorSubcoreMesh(core_axis_name="c", subcore_axis_name="s")
    scratch = (
        [pltpu.VMEM((n_cols, rpw), jnp.int32)]
        + [pltpu.VMEM((_RCHUNK, D_MODEL), jnp.float32) for _ in range(_NBUF)]
        + [
            pltpu.VMEM((8, _RCHUNK // 128, 12, 129), jnp.float32)
            for _ in range(_NBUF)
        ]
        + [pltpu.SemaphoreType.DMA for _ in range(2 * _NBUF)]
    )
    fn = pl.kernel(
        _emb_body,
        out_type=jax.ShapeDtypeStruct((n_cols, 8, nrt, 8, 128), jnp.float32),
        mesh=mesh,
        scratch_types=scratch,
        compiler_params=pltpu.CompilerParams(
            use_tc_tiling_on_sc=False, needs_layout_passes=False
        ),
    )
    o5 = fn(table, xt)
    out_t = o5.transpose(0, 1, 3, 2, 4).reshape(n_cols, D_MODEL, n_rows)
    return out_t.transpose(2, 0, 1)
